# baseline (speedup 1.0000x reference)
import functools

import jax
import jax.numpy as jnp
from jax import lax
from jax.experimental import pallas as pl
from jax.experimental.pallas import tpu as pltpu
from jax.experimental.pallas import tpu_sc as plsc

N = 65536
D = 256

_INFO = plsc.get_sparse_core_info()
_NC = _INFO.num_cores
_NS = _INFO.num_subcores
_NW = _NC * _NS
_CPW = N // _NW
_NV = _CPW // 16
_SHARE = N // _NS
_RADIX = 256
_CHUNK = 128

_SC_PARAMS = pltpu.CompilerParams(use_tc_tiling_on_sc=False,
                                  needs_layout_passes=False)


def _mesh():
    return plsc.VectorSubcoreMesh(core_axis_name="c", subcore_axis_name="s")


def _iota16():
    return lax.iota(jnp.int32, 16)


def _srl(v, s):
    return lax.shift_right_logical(v, jnp.full((16,), s, jnp.int32))


def _prep_body(a_ref, l2_ref, r_ref, atom_ref, pts_ref, pv_ref, av_ref,
               ex_ref, mkey_ref, hist_ref, packed_ref):
    score = 0.5 * jnp.mean(a_ref[...], axis=1) + 0.5 * l2_ref[...]
    u = lax.bitcast_convert_type(-r_ref[...], jnp.int32)
    m = jnp.where(u < 0, ~u, u ^ jnp.int32(-2147483648))
    mkey_ref[...] = m
    d = m & 255
    bins = lax.broadcasted_iota(jnp.int32, (1, _RADIX), 1)
    hist_ref[...] = jnp.sum((d[:, None] == bins).astype(jnp.int32),
                            axis=0).reshape(1, 1, _RADIX)
    packed_ref[...] = jnp.concatenate([
        lax.bitcast_convert_type(atom_ref[...], jnp.float32)[:, None],
        pts_ref[...],
        pv_ref[...][:, None],
        av_ref[...],
        score[:, None],
        ex_ref[...][:, None],
        r_ref[...][:, None],
        jnp.zeros((score.shape[0], 5), jnp.float32),
    ], axis=1)


def _prep(alignment_tensor, l2_alignment, react, atom, pts, pv, av, ex):
    R = 2048
    vec = lambda: pl.BlockSpec((R,), lambda i: (i,))
    return pl.pallas_call(
        _prep_body,
        grid=(N // R,),
        in_specs=[
            pl.BlockSpec((R, D), lambda i: (i, 0)),
            vec(), vec(), vec(),
            pl.BlockSpec((R, 3), lambda i: (i, 0)),
            vec(),
            pl.BlockSpec((R, 3), lambda i: (i, 0)),
            vec(),
        ],
        out_specs=[vec(),
                   pl.BlockSpec((1, 1, _RADIX), lambda i: (i, 0, 0)),
                   pl.BlockSpec((R, 16), lambda i: (i, 0))],
        out_shape=[jax.ShapeDtypeStruct((N,), jnp.int32),
                   jax.ShapeDtypeStruct((_NW, 1, _RADIX), jnp.int32),
                   jax.ShapeDtypeStruct((N, 16), jnp.float32)],
    )(alignment_tensor, l2_alignment, react, atom, pts, pv, av, ex)


def _pass_kernel(shift, first, last):
    nxt = shift + 8

    scratch = [
        pltpu.VMEM((_NW, _RADIX), jnp.int32),
        pltpu.VMEM((_NW, _RADIX), jnp.int32),
        pltpu.VMEM((_CPW,), jnp.int32),
        pltpu.VMEM((_CPW,), jnp.int32),
        pltpu.VMEM((_CPW,), jnp.int32),
        pltpu.VMEM((_CPW,), jnp.int32),
        pltpu.VMEM((_RADIX,), jnp.int32),
        pltpu.VMEM((_CPW,), jnp.int32),
        pltpu.VMEM((_CPW,), jnp.int32),
        pltpu.VMEM((_CPW // _CHUNK, _CHUNK), jnp.int32),
        pltpu.SemaphoreType.DMA,
    ]
    if last:
        scratch.append(pltpu.VMEM((_CPW, 16), jnp.float32))
        out_type = jax.ShapeDtypeStruct((N, 16), jnp.float32)
    else:
        scratch += [
            pltpu.VMEM((_SHARE,), jnp.int32),
            pltpu.VMEM((_SHARE,), jnp.int32),
            pltpu.VMEM((2 * 16 * _RADIX,), jnp.int32),
            pltpu.VMEM((2, _RADIX), jnp.int32),
            pltpu.VMEM_SHARED((N,), jnp.int32),
            pltpu.VMEM_SHARED((N,), jnp.int32),
        ]
        out_type = (jax.ShapeDtypeStruct((_NC, N), jnp.int32),
                    jax.ShapeDtypeStruct((_NC, N), jnp.int32),
                    jax.ShapeDtypeStruct((_NC, _NW, _RADIX), jnp.int32))

    def k(*refs):
        if first:
            k_hbm, h_hbm = refs[:2]
            refs = refs[2:]
        else:
            k_hbm, i_hbm, h_hbm = refs[:3]
            refs = refs[3:]
        if last:
            t_hbm, rows_hbm = refs[:2]
            (hist_v, hist2_v, kc0, kc1, ic0, ic1, counter_v, kmerged,
             imerged, pos2, sem, rows_v) = refs[2:]
        else:
            ko_hbm, io_hbm, hp_hbm = refs[:3]
            (hist_v, hist2_v, kc0, kc1, ic0, ic1, counter_v, kmerged,
             imerged, pos2, sem, islice_v, kslice_v, c2_v, row_v,
             kshared, ishared) = refs[3:]

        cc = lax.axis_index("c")
        ss = lax.axis_index("s")
        wid = ss * _NC + cc
        sl = pl.ds(wid * _CPW, _CPW)
        zero = jnp.zeros((16,), jnp.int32)
        iota = _iota16()

        if first:
            pltpu.sync_copy(h_hbm, hist_v)
            pltpu.sync_copy(k_hbm.at[sl], kc0)
        else:
            pltpu.sync_copy(h_hbm.at[0], hist_v)
            pltpu.sync_copy(h_hbm.at[1], hist2_v)
            pltpu.sync_copy(k_hbm.at[0, sl], kc0)
            pltpu.sync_copy(k_hbm.at[1, sl], kc1)
            pltpu.sync_copy(i_hbm.at[0, sl], ic0)
            pltpu.sync_copy(i_hbm.at[1, sl], ic1)

        if not last:
            negones = jnp.full((16,), -1, jnp.int32)

            def nbody(i, c):
                islice_v[pl.ds(pl.multiple_of(i * 16, 16), 16)] = negones
                return c

            lax.fori_loop(0, _SHARE // 16, nbody, 0)
            my_slice = pl.ds(ss * _SHARE, _SHARE)
            pltpu.sync_copy(islice_v, ishared.at[my_slice])

        carry = jnp.int32(0)
        for g in range(_RADIX // 16):
            def tbody(t, c):
                accg, myg = c
                h = hist_v[t, pl.ds(g * 16, 16)]
                if not first:
                    h = h + hist2_v[t, pl.ds(g * 16, 16)]
                myg = jnp.where(t == wid, accg, myg)
                return accg + h, myg

            accg, myg = lax.fori_loop(0, _NW, tbody, (zero, zero))
            cs = plsc.cumsum(accg)
            offg = (cs - accg) + myg + jnp.broadcast_to(carry, (16,))
            counter_v[pl.ds(g * 16, 16)] = offg
            carry = carry + jnp.sum(accg)

        if not last:
            plsc.subcore_barrier()

        def body(v, c):
            off = pl.ds(pl.multiple_of(v * 16, 16), 16)
            if first:
                k16 = kc0[off]
                i16 = wid * _CPW + v * 16 + iota
            else:
                i1 = ic1[off]
                msel = i1 >= 0
                k16 = jnp.where(msel, kc1[off], kc0[off])
                i16 = jnp.where(msel, i1, ic0[off])
            d = _srl(k16, shift) & 255
            cnt, lastm = plsc.scan_count(d)
            bases = plsc.load_gather(counter_v, [d])
            pos = bases + cnt - 1
            plsc.store_scatter(counter_v, [d], pos + 1, mask=lastm)
            kmerged[off] = k16
            imerged[off] = i16
            row = lax.shift_right_logical(v, 3)
            col = (v & 7) * 16
            pos2[row, pl.ds(pl.multiple_of(col, 16), 16)] = pos
            return c

        lax.fori_loop(0, _NV, body, 0)

        if last:
            descs = []
            for j in range(_CPW // _CHUNK):
                rsl = pl.ds(j * _CHUNK, _CHUNK)
                descs.append(pltpu.async_copy(
                    t_hbm.at[imerged.at[rsl]], rows_v.at[rsl], sem))
            for dd in descs:
                dd.wait()
            descs = []
            for j in range(_CPW // _CHUNK):
                rsl = pl.ds(j * _CHUNK, _CHUNK)
                descs.append(pltpu.async_copy(
                    rows_v.at[rsl], rows_hbm.at[pos2.at[j]], sem))
            for dd in descs:
                dd.wait()
            return

        descs = []
        for j in range(_CPW // _CHUNK):
            sj = pl.ds(j * _CHUNK, _CHUNK)
            descs.append(pltpu.async_copy(kmerged.at[sj],
                                          kshared.at[pos2.at[j]], sem))
            descs.append(pltpu.async_copy(imerged.at[sj],
                                          ishared.at[pos2.at[j]], sem))
        for dd in descs:
            dd.wait()
        plsc.subcore_barrier()

        pltpu.sync_copy(kshared.at[my_slice], kslice_v)
        pltpu.sync_copy(ishared.at[my_slice], islice_v)
        pltpu.sync_copy(kslice_v, ko_hbm.at[cc, my_slice])
        pltpu.sync_copy(islice_v, io_hbm.at[cc, my_slice])

        def zbody(i, c):
            c2_v[pl.ds(pl.multiple_of(i * 16, 16), 16)] = zero
            return c

        lax.fori_loop(0, 2 * 16 * _RADIX // 16, zbody, 0)
        lanebase = iota * _RADIX
        ones = jnp.ones((16,), jnp.int32)

        def hbody(v, c):
            off = pl.ds(pl.multiple_of(v * 16, 16), 16)
            k16 = kslice_v[off]
            i16 = islice_v[off]
            valid = i16 >= 0
            d = _srl(k16, nxt) & 255
            half = lax.shift_right_logical(v, 7)
            plsc.addupdate_scatter(c2_v, [half * (16 * _RADIX) + lanebase + d],
                                   ones, mask=valid)
            return c

        lax.fori_loop(0, _SHARE // 16, hbody, 0)

        def rbody(i, c):
            h = lax.shift_right_logical(i, 4)
            g = i & 15

            def sbody(l, acc):
                return acc + c2_v[pl.ds(pl.multiple_of(
                    h * (16 * _RADIX) + l * _RADIX + g * 16, 16), 16)]

            acc = lax.fori_loop(0, 16, sbody, zero)
            row_v[h, pl.ds(pl.multiple_of(g * 16, 16), 16)] = acc
            return c

        lax.fori_loop(0, 32, rbody, 0)
        pltpu.sync_copy(row_v, hp_hbm.at[cc, pl.ds(ss * 2, 2)])

    kern = functools.partial(
        pl.kernel, mesh=_mesh(), out_type=out_type,
        compiler_params=_SC_PARAMS, scratch_types=scratch)(k)
    return kern


def _sc_sort_and_rank(mkeys, hist0, table):
    p0 = _pass_kernel(0, first=True, last=False)
    k2, i2, hp = p0(mkeys, hist0)
    p1 = _pass_kernel(8, first=False, last=False)
    k2, i2, hp = p1(k2, i2, hp)
    p2 = _pass_kernel(16, first=False, last=False)
    k2, i2, hp = p2(k2, i2, hp)
    p3 = _pass_kernel(24, first=False, last=True)
    return p3(k2, i2, hp, table)


def kernel(alignment_tensor, l2_alignment, effective_reactivity, atom_indices,
           refined_peak_points, refined_peak_values, approach_vectors,
           exposure_scores):
    mkeys, hist0, packed = _prep(
        alignment_tensor, l2_alignment, effective_reactivity, atom_indices,
        refined_peak_points, refined_peak_values, approach_vectors,
        exposure_scores)
    hist0 = hist0.reshape(_NW, _RADIX)

    rows = _sc_sort_and_rank(mkeys, hist0, packed)

    ranked_atom_indices = lax.bitcast_convert_type(rows[:, 0], jnp.int32)
    som_coordinates = rows[:, 1:4]
    psi_peak = rows[:, 4]
    approach_vector = rows[:, 5:8]
    alignment_score_ranked = rows[:, 8]
    exposure_score = rows[:, 9]
    effective_reactivity_ranked = rows[:, 10]
    return (ranked_atom_indices, som_coordinates, psi_peak, approach_vector,
            alignment_score_ranked, exposure_score, effective_reactivity_ranked)


# speedup vs baseline: 1.1913x; 1.1913x over previous
import functools

import jax
import jax.numpy as jnp
from jax import lax
from jax.experimental import pallas as pl
from jax.experimental.pallas import tpu as pltpu
from jax.experimental.pallas import tpu_sc as plsc

N = 65536
D = 256

_INFO = plsc.get_sparse_core_info()
_NC = _INFO.num_cores
_NS = _INFO.num_subcores
_NW = _NC * _NS
_CPW = N // _NW
_NV = _CPW // 16
_SHARE = N // _NS
_RADIX = 256
_CHUNK = 128

_SC_PARAMS = pltpu.CompilerParams(use_tc_tiling_on_sc=False,
                                  needs_layout_passes=False)


def _mesh():
    return plsc.VectorSubcoreMesh(core_axis_name="c", subcore_axis_name="s")


def _iota16():
    return lax.iota(jnp.int32, 16)


def _srl(v, s):
    return lax.shift_right_logical(v, jnp.full((16,), s, jnp.int32))


def _prep_body(a_ref, l2_ref, r_ref, score_ref, mkey_ref, hist_ref):
    score_ref[...] = 0.5 * jnp.mean(a_ref[...], axis=1) + 0.5 * l2_ref[...]
    u = lax.bitcast_convert_type(-r_ref[...], jnp.int32)
    m = jnp.where(u < 0, ~u, u ^ jnp.int32(-2147483648))
    mkey_ref[...] = m
    d = m & 255
    bins = lax.broadcasted_iota(jnp.int32, (1, _RADIX), 1)
    hist_ref[...] = jnp.sum((d[:, None] == bins).astype(jnp.int32),
                            axis=0).reshape(1, 1, _RADIX)


def _prep(alignment_tensor, l2_alignment, react):
    R = 2048
    vec = lambda: pl.BlockSpec((R,), lambda i: (i,))
    return pl.pallas_call(
        _prep_body,
        grid=(N // R,),
        in_specs=[pl.BlockSpec((R, D), lambda i: (i, 0)), vec(), vec()],
        out_specs=[vec(), vec(),
                   pl.BlockSpec((1, 1, _RADIX), lambda i: (i, 0, 0))],
        out_shape=[jax.ShapeDtypeStruct((N,), jnp.float32),
                   jax.ShapeDtypeStruct((N,), jnp.int32),
                   jax.ShapeDtypeStruct((_NW, 1, _RADIX), jnp.int32)],
    )(alignment_tensor, l2_alignment, react)


def _pass_kernel(shift, first, last):
    nxt = shift + 8

    scratch = [
        pltpu.VMEM((_NW, _RADIX), jnp.int32),
        pltpu.VMEM((_NW, _RADIX), jnp.int32),
        pltpu.VMEM((_CPW,), jnp.int32),
        pltpu.VMEM((_CPW,), jnp.int32),
        pltpu.VMEM((_CPW,), jnp.int32),
        pltpu.VMEM((_CPW,), jnp.int32),
        pltpu.VMEM((_RADIX,), jnp.int32),
        pltpu.VMEM((_CPW,), jnp.int32),
        pltpu.VMEM((_CPW,), jnp.int32),
        pltpu.VMEM((_CPW // _CHUNK, _CHUNK), jnp.int32),
        pltpu.VMEM((_SHARE,), jnp.int32),
        pltpu.VMEM((_SHARE,), jnp.int32),
        pltpu.VMEM((2 * 16 * _RADIX,), jnp.int32),
        pltpu.VMEM((2, _RADIX), jnp.int32),
        pltpu.VMEM_SHARED((N,), jnp.int32),
        pltpu.VMEM_SHARED((N,), jnp.int32),
        pltpu.SemaphoreType.DMA,
    ]
    if last:
        out_type = jax.ShapeDtypeStruct((_NC, N), jnp.int32)
    else:
        out_type = (jax.ShapeDtypeStruct((_NC, N), jnp.int32),
                    jax.ShapeDtypeStruct((_NC, N), jnp.int32),
                    jax.ShapeDtypeStruct((_NC, _NW, _RADIX), jnp.int32))

    def k(*refs):
        if first:
            k_hbm, h_hbm = refs[:2]
            refs = refs[2:]
        else:
            k_hbm, i_hbm, h_hbm = refs[:3]
            refs = refs[3:]
        if last:
            io_hbm = refs[0]
            refs = refs[1:]
        else:
            ko_hbm, io_hbm, hp_hbm = refs[:3]
            refs = refs[3:]
        (hist_v, hist2_v, kc0, kc1, ic0, ic1, counter_v, kmerged, imerged,
         pos2, islice_v, kslice_v, c2_v, row_v, kshared, ishared, sem) = refs

        cc = lax.axis_index("c")
        ss = lax.axis_index("s")
        wid = ss * _NC + cc
        sl = pl.ds(wid * _CPW, _CPW)
        zero = jnp.zeros((16,), jnp.int32)
        iota = _iota16()

        if first:
            pltpu.sync_copy(h_hbm, hist_v)
            pltpu.sync_copy(k_hbm.at[sl], kc0)
        else:
            pltpu.sync_copy(h_hbm.at[0], hist_v)
            pltpu.sync_copy(h_hbm.at[1], hist2_v)
            pltpu.sync_copy(k_hbm.at[0, sl], kc0)
            pltpu.sync_copy(k_hbm.at[1, sl], kc1)
            pltpu.sync_copy(i_hbm.at[0, sl], ic0)
            pltpu.sync_copy(i_hbm.at[1, sl], ic1)

        negones = jnp.full((16,), -1, jnp.int32)

        def nbody(i, c):
            islice_v[pl.ds(pl.multiple_of(i * 16, 16), 16)] = negones
            return c

        lax.fori_loop(0, _SHARE // 16, nbody, 0)
        my_slice = pl.ds(ss * _SHARE, _SHARE)
        pltpu.sync_copy(islice_v, ishared.at[my_slice])

        def gbody(g, carry):
            go = pl.ds(pl.multiple_of(g * 16, 16), 16)

            def tbody(t, c):
                accg, myg = c
                h = hist_v[t, go]
                if not first:
                    h = h + hist2_v[t, go]
                myg = jnp.where(t == wid, accg, myg)
                return accg + h, myg

            accg, myg = lax.fori_loop(0, _NW, tbody, (zero, zero))
            cs = plsc.cumsum(accg)
            counter_v[go] = (cs - accg) + myg + jnp.broadcast_to(carry, (16,))
            return carry + jnp.sum(accg)

        lax.fori_loop(0, _RADIX // 16, gbody, jnp.int32(0))

        plsc.subcore_barrier()

        def body(v, c):
            off = pl.ds(pl.multiple_of(v * 16, 16), 16)
            if first:
                k16 = kc0[off]
                i16 = wid * _CPW + v * 16 + iota
            else:
                i1 = ic1[off]
                msel = i1 >= 0
                k16 = jnp.where(msel, kc1[off], kc0[off])
                i16 = jnp.where(msel, i1, ic0[off])
            d = _srl(k16, shift) & 255
            cnt, lastm = plsc.scan_count(d)
            bases = plsc.load_gather(counter_v, [d])
            pos = bases + cnt - 1
            plsc.store_scatter(counter_v, [d], pos + 1, mask=lastm)
            kmerged[off] = k16
            imerged[off] = i16
            row = lax.shift_right_logical(v, 3)
            col = (v & 7) * 16
            pos2[row, pl.ds(pl.multiple_of(col, 16), 16)] = pos
            return c

        lax.fori_loop(0, _NV, body, 0)

        descs = []
        for j in range(_CPW // _CHUNK):
            sj = pl.ds(j * _CHUNK, _CHUNK)
            if not last:
                descs.append(pltpu.async_copy(kmerged.at[sj],
                                              kshared.at[pos2.at[j]], sem))
            descs.append(pltpu.async_copy(imerged.at[sj],
                                          ishared.at[pos2.at[j]], sem))
        for dd in descs:
            dd.wait()
        plsc.subcore_barrier()

        pltpu.sync_copy(ishared.at[my_slice], islice_v)
        pltpu.sync_copy(islice_v, io_hbm.at[cc, my_slice])
        if last:
            return
        pltpu.sync_copy(kshared.at[my_slice], kslice_v)
        pltpu.sync_copy(kslice_v, ko_hbm.at[cc, my_slice])

        def zbody(i, c):
            c2_v[pl.ds(pl.multiple_of(i * 16, 16), 16)] = zero
            return c

        lax.fori_loop(0, 2 * 16 * _RADIX // 16, zbody, 0)
        lanebase = iota * _RADIX
        ones = jnp.ones((16,), jnp.int32)

        def hbody(v, c):
            off = pl.ds(pl.multiple_of(v * 16, 16), 16)
            k16 = kslice_v[off]
            i16 = islice_v[off]
            valid = i16 >= 0
            d = _srl(k16, nxt) & 255
            half = lax.shift_right_logical(v, 7)
            plsc.addupdate_scatter(c2_v, [half * (16 * _RADIX) + lanebase + d],
                                   ones, mask=valid)
            return c

        lax.fori_loop(0, _SHARE // 16, hbody, 0)

        def rbody(i, c):
            h = lax.shift_right_logical(i, 4)
            g = i & 15

            def sbody(l, acc):
                return acc + c2_v[pl.ds(pl.multiple_of(
                    h * (16 * _RADIX) + l * _RADIX + g * 16, 16), 16)]

            acc = lax.fori_loop(0, 16, sbody, zero)
            row_v[h, pl.ds(pl.multiple_of(g * 16, 16), 16)] = acc
            return c

        lax.fori_loop(0, 32, rbody, 0)
        pltpu.sync_copy(row_v, hp_hbm.at[cc, pl.ds(ss * 2, 2)])

    kern = functools.partial(
        pl.kernel, mesh=_mesh(), out_type=out_type,
        compiler_params=_SC_PARAMS, scratch_types=scratch)(k)
    return kern


def _gather_kernel(i2, atom, pts, pv, av, score, ex, react):
    out_type = (jax.ShapeDtypeStruct((N,), jnp.int32),
                jax.ShapeDtypeStruct((N, 3), jnp.float32),
                jax.ShapeDtypeStruct((N,), jnp.float32),
                jax.ShapeDtypeStruct((N, 3), jnp.float32),
                jax.ShapeDtypeStruct((N,), jnp.float32),
                jax.ShapeDtypeStruct((N,), jnp.float32),
                jax.ShapeDtypeStruct((N,), jnp.float32))

    @functools.partial(
        pl.kernel,
        mesh=_mesh(),
        out_type=out_type,
        compiler_params=_SC_PARAMS,
        scratch_types=[
            pltpu.VMEM((_CPW,), jnp.int32),
            pltpu.VMEM((_CPW,), jnp.int32),
            pltpu.VMEM((_CPW,), jnp.int32),
            pltpu.VMEM((_CPW,), jnp.int32),
            pltpu.VMEM((_CPW, 3), jnp.float32),
            pltpu.VMEM((_CPW,), jnp.float32),
            pltpu.VMEM((_CPW, 3), jnp.float32),
            pltpu.VMEM((_CPW,), jnp.float32),
            pltpu.VMEM((_CPW,), jnp.float32),
            pltpu.VMEM((_CPW,), jnp.float32),
            pltpu.SemaphoreType.DMA,
        ],
    )
    def k(i_hbm, atom_hbm, pts_hbm, pv_hbm, av_hbm, sc_hbm, ex_hbm, re_hbm,
          o_atom, o_pts, o_pv, o_av, o_sc, o_ex, o_re,
          ic0, ic1, ord_v, g_atom, g_pts, g_pv, g_av, g_sc, g_ex, g_re, sem):
        cc = lax.axis_index("c")
        ss = lax.axis_index("s")
        wid = ss * _NC + cc
        sl = pl.ds(wid * _CPW, _CPW)
        pltpu.sync_copy(i_hbm.at[0, sl], ic0)
        pltpu.sync_copy(i_hbm.at[1, sl], ic1)

        def mbody(v, c):
            off = pl.ds(pl.multiple_of(v * 16, 16), 16)
            b = ic1[off]
            ord_v[off] = jnp.where(b >= 0, b, ic0[off])
            return c

        lax.fori_loop(0, _NV, mbody, 0)

        descs = [
            pltpu.async_copy(atom_hbm.at[ord_v], g_atom, sem),
            pltpu.async_copy(pts_hbm.at[ord_v], g_pts, sem),
            pltpu.async_copy(pv_hbm.at[ord_v], g_pv, sem),
            pltpu.async_copy(av_hbm.at[ord_v], g_av, sem),
            pltpu.async_copy(sc_hbm.at[ord_v], g_sc, sem),
            pltpu.async_copy(ex_hbm.at[ord_v], g_ex, sem),
            pltpu.async_copy(re_hbm.at[ord_v], g_re, sem),
        ]
        for dd in descs:
            dd.wait()

        pltpu.sync_copy(g_atom, o_atom.at[sl])
        pltpu.sync_copy(g_pts, o_pts.at[sl])
        pltpu.sync_copy(g_pv, o_pv.at[sl])
        pltpu.sync_copy(g_av, o_av.at[sl])
        pltpu.sync_copy(g_sc, o_sc.at[sl])
        pltpu.sync_copy(g_ex, o_ex.at[sl])
        pltpu.sync_copy(g_re, o_re.at[sl])

    return k(i2, atom, pts, pv, av, score, ex, react)


def kernel(alignment_tensor, l2_alignment, effective_reactivity, atom_indices,
           refined_peak_points, refined_peak_values, approach_vectors,
           exposure_scores):
    score, mkeys, hist0 = _prep(alignment_tensor, l2_alignment,
                                effective_reactivity)
    hist0 = hist0.reshape(_NW, _RADIX)

    p0 = _pass_kernel(0, first=True, last=False)
    k2, i2, hp = p0(mkeys, hist0)
    p1 = _pass_kernel(8, first=False, last=False)
    k2, i2, hp = p1(k2, i2, hp)
    p2 = _pass_kernel(16, first=False, last=False)
    k2, i2, hp = p2(k2, i2, hp)
    p3 = _pass_kernel(24, first=False, last=True)
    i2 = p3(k2, i2, hp)

    return _gather_kernel(i2, atom_indices, refined_peak_points,
                          refined_peak_values, approach_vectors, score,
                          exposure_scores, effective_reactivity)


# speedup vs baseline: 1.5817x; 1.3276x over previous
import functools

import jax
import jax.numpy as jnp
from jax import lax
from jax.experimental import pallas as pl
from jax.experimental.pallas import tpu as pltpu
from jax.experimental.pallas import tpu_sc as plsc

N = 65536
D = 256

_INFO = plsc.get_sparse_core_info()
_NC = _INFO.num_cores
_NS = _INFO.num_subcores
_NW = _NC * _NS
_CPW = N // _NW
_NV = _CPW // 16
_SHARE = N // _NS
_RADIX = 256
_CHUNK = 128

_SC_PARAMS = pltpu.CompilerParams(use_tc_tiling_on_sc=False,
                                  needs_layout_passes=False)


def _mesh():
    return plsc.VectorSubcoreMesh(core_axis_name="c", subcore_axis_name="s")


def _iota16():
    return lax.iota(jnp.int32, 16)


def _srl(v, s):
    return lax.shift_right_logical(v, jnp.full((16,), s, jnp.int32))


def _prep_body(a_ref, l2_ref, r_ref, score_ref, mkey_ref, hist_ref):
    score_ref[...] = 0.5 * jnp.mean(a_ref[...], axis=1) + 0.5 * l2_ref[...]
    u = lax.bitcast_convert_type(-r_ref[...], jnp.int32)
    m = jnp.where(u < 0, ~u, u ^ jnp.int32(-2147483648))
    mkey_ref[...] = m
    d = m & 255
    bins = lax.broadcasted_iota(jnp.int32, (1, _RADIX), 1)
    hist_ref[...] = jnp.sum((d[:, None] == bins).astype(jnp.int32),
                            axis=0).reshape(1, 1, _RADIX)


def _prep(alignment_tensor, l2_alignment, react):
    R = 2048
    vec = lambda: pl.BlockSpec((R,), lambda i: (i,))
    return pl.pallas_call(
        _prep_body,
        grid=(N // R,),
        in_specs=[pl.BlockSpec((R, D), lambda i: (i, 0)), vec(), vec()],
        out_specs=[vec(), vec(),
                   pl.BlockSpec((1, 1, _RADIX), lambda i: (i, 0, 0))],
        out_shape=[jax.ShapeDtypeStruct((N,), jnp.float32),
                   jax.ShapeDtypeStruct((N,), jnp.int32),
                   jax.ShapeDtypeStruct((_NW, 1, _RADIX), jnp.int32)],
    )(alignment_tensor, l2_alignment, react)


def _pass_kernel(shift, first, last):
    nxt = shift + 8

    scratch = [
        pltpu.VMEM((_NW, _RADIX), jnp.int32),
        pltpu.VMEM((_NW, _RADIX), jnp.int32),
        pltpu.VMEM((_CPW,), jnp.int32),
        pltpu.VMEM((_CPW,), jnp.int32),
        pltpu.VMEM((_CPW,), jnp.int32),
        pltpu.VMEM((_CPW,), jnp.int32),
        pltpu.VMEM((_RADIX,), jnp.int32),
        pltpu.VMEM((_CPW,), jnp.int32),
        pltpu.VMEM((_CPW,), jnp.int32),
        pltpu.VMEM((_CPW // _CHUNK, _CHUNK), jnp.int32),
        pltpu.VMEM((_SHARE,), jnp.int32),
        pltpu.VMEM((_SHARE,), jnp.int32),
        pltpu.VMEM((2 * 16 * _RADIX,), jnp.int32),
        pltpu.VMEM((2, _RADIX), jnp.int32),
        pltpu.VMEM_SHARED((N,), jnp.int32),
        pltpu.VMEM_SHARED((N,), jnp.int32),
        pltpu.SemaphoreType.DMA,
    ]
    if last:
        out_type = jax.ShapeDtypeStruct((_NC, N), jnp.int32)
    else:
        out_type = (jax.ShapeDtypeStruct((_NC, N), jnp.int32),
                    jax.ShapeDtypeStruct((_NC, N), jnp.int32),
                    jax.ShapeDtypeStruct((_NC, _NW, _RADIX), jnp.int32))

    def k(*refs):
        if first:
            k_hbm, h_hbm = refs[:2]
            refs = refs[2:]
        else:
            k_hbm, i_hbm, h_hbm = refs[:3]
            refs = refs[3:]
        if last:
            io_hbm = refs[0]
            refs = refs[1:]
        else:
            ko_hbm, io_hbm, hp_hbm = refs[:3]
            refs = refs[3:]
        (hist_v, hist2_v, kc0, kc1, ic0, ic1, counter_v, kmerged, imerged,
         pos2, islice_v, kslice_v, c2_v, row_v, kshared, ishared, sem) = refs

        cc = lax.axis_index("c")
        ss = lax.axis_index("s")
        wid = ss * _NC + cc
        sl = pl.ds(wid * _CPW, _CPW)
        zero = jnp.zeros((16,), jnp.int32)
        iota = _iota16()

        if first:
            pltpu.sync_copy(h_hbm, hist_v)
            pltpu.sync_copy(k_hbm.at[sl], kc0)
        else:
            pltpu.sync_copy(h_hbm.at[0], hist_v)
            pltpu.sync_copy(h_hbm.at[1], hist2_v)
            pltpu.sync_copy(k_hbm.at[0, sl], kc0)
            pltpu.sync_copy(k_hbm.at[1, sl], kc1)
            pltpu.sync_copy(i_hbm.at[0, sl], ic0)
            pltpu.sync_copy(i_hbm.at[1, sl], ic1)

        negones = jnp.full((16,), -1, jnp.int32)

        def nbody(i, c):
            islice_v[pl.ds(pl.multiple_of(i * 16, 16), 16)] = negones
            return c

        lax.fori_loop(0, _SHARE // 16, nbody, 0)
        my_slice = pl.ds(ss * _SHARE, _SHARE)
        pltpu.sync_copy(islice_v, ishared.at[my_slice])

        def gbody(g, carry):
            go = pl.ds(pl.multiple_of(g * 16, 16), 16)

            def tbody(t, c):
                accg, myg = c
                h = hist_v[t, go]
                if not first:
                    h = h + hist2_v[t, go]
                myg = jnp.where(t == wid, accg, myg)
                return accg + h, myg

            accg, myg = lax.fori_loop(0, _NW, tbody, (zero, zero))
            cs = plsc.cumsum(accg)
            counter_v[go] = (cs - accg) + myg + jnp.broadcast_to(carry, (16,))
            return carry + jnp.sum(accg)

        lax.fori_loop(0, _RADIX // 16, gbody, jnp.int32(0))

        plsc.subcore_barrier()

        def body(v, c):
            off = pl.ds(pl.multiple_of(v * 16, 16), 16)
            if first:
                k16 = kc0[off]
                i16 = wid * _CPW + v * 16 + iota
            else:
                i1 = ic1[off]
                msel = i1 >= 0
                k16 = jnp.where(msel, kc1[off], kc0[off])
                i16 = jnp.where(msel, i1, ic0[off])
            d = _srl(k16, shift) & 255
            cnt, lastm = plsc.scan_count(d)
            bases = plsc.load_gather(counter_v, [d])
            pos = bases + cnt - 1
            plsc.store_scatter(counter_v, [d], pos + 1, mask=lastm)
            kmerged[off] = k16
            imerged[off] = i16
            row = lax.shift_right_logical(v, 3)
            col = (v & 7) * 16
            pos2[row, pl.ds(pl.multiple_of(col, 16), 16)] = pos
            return c

        lax.fori_loop(0, _NV, body, 0)

        descs = []
        for j in range(_CPW // _CHUNK):
            sj = pl.ds(j * _CHUNK, _CHUNK)
            if not last:
                descs.append(pltpu.async_copy(kmerged.at[sj],
                                              kshared.at[pos2.at[j]], sem))
            descs.append(pltpu.async_copy(imerged.at[sj],
                                          ishared.at[pos2.at[j]], sem))
        for dd in descs:
            dd.wait()
        plsc.subcore_barrier()

        pltpu.sync_copy(ishared.at[my_slice], islice_v)
        pltpu.sync_copy(islice_v, io_hbm.at[cc, my_slice])
        if last:
            return
        pltpu.sync_copy(kshared.at[my_slice], kslice_v)
        pltpu.sync_copy(kslice_v, ko_hbm.at[cc, my_slice])

        def zbody(i, c):
            c2_v[pl.ds(pl.multiple_of(i * 16, 16), 16)] = zero
            return c

        lax.fori_loop(0, 2 * 16 * _RADIX // 16, zbody, 0)
        lanebase = iota * _RADIX
        ones = jnp.ones((16,), jnp.int32)

        def hbody(v, c):
            off = pl.ds(pl.multiple_of(v * 16, 16), 16)
            k16 = kslice_v[off]
            i16 = islice_v[off]
            valid = i16 >= 0
            d = _srl(k16, nxt) & 255
            half = lax.shift_right_logical(v, 7)
            plsc.addupdate_scatter(c2_v, [half * (16 * _RADIX) + lanebase + d],
                                   ones, mask=valid)
            return c

        lax.fori_loop(0, _SHARE // 16, hbody, 0)

        def rbody(i, c):
            h = lax.shift_right_logical(i, 4)
            g = i & 15

            def sbody(l, acc):
                return acc + c2_v[pl.ds(pl.multiple_of(
                    h * (16 * _RADIX) + l * _RADIX + g * 16, 16), 16)]

            acc = lax.fori_loop(0, 16, sbody, zero)
            row_v[h, pl.ds(pl.multiple_of(g * 16, 16), 16)] = acc
            return c

        lax.fori_loop(0, 32, rbody, 0)
        pltpu.sync_copy(row_v, hp_hbm.at[cc, pl.ds(ss * 2, 2)])

    kern = functools.partial(
        pl.kernel, mesh=_mesh(), out_type=out_type,
        compiler_params=_SC_PARAMS, scratch_types=scratch)(k)
    return kern


def _gather_kernel(i2, atom, pv, score, ex, react):
    out_type = (jax.ShapeDtypeStruct((N,), jnp.int32),
                jax.ShapeDtypeStruct((N,), jnp.float32),
                jax.ShapeDtypeStruct((N,), jnp.float32),
                jax.ShapeDtypeStruct((N,), jnp.float32),
                jax.ShapeDtypeStruct((N,), jnp.float32),
                jax.ShapeDtypeStruct((N,), jnp.int32))

    @functools.partial(
        pl.kernel,
        mesh=_mesh(),
        out_type=out_type,
        compiler_params=_SC_PARAMS,
        scratch_types=[
            pltpu.VMEM((_CPW,), jnp.int32),
            pltpu.VMEM((_CPW,), jnp.int32),
            pltpu.VMEM((_CPW,), jnp.int32),
            pltpu.VMEM((_CPW,), jnp.int32),
            pltpu.VMEM((_CPW,), jnp.float32),
            pltpu.VMEM((_CPW,), jnp.float32),
            pltpu.VMEM((_CPW,), jnp.float32),
            pltpu.VMEM((_CPW,), jnp.float32),
            pltpu.SemaphoreType.DMA,
        ],
    )
    def k(i_hbm, atom_hbm, pv_hbm, sc_hbm, ex_hbm, re_hbm,
          o_atom, o_pv, o_sc, o_ex, o_re, o_ord,
          ic0, ic1, ord_v, g_atom, g_pv, g_sc, g_ex, g_re, sem):
        cc = lax.axis_index("c")
        ss = lax.axis_index("s")
        wid = ss * _NC + cc
        sl = pl.ds(wid * _CPW, _CPW)
        pltpu.sync_copy(i_hbm.at[0, sl], ic0)
        pltpu.sync_copy(i_hbm.at[1, sl], ic1)

        def mbody(v, c):
            off = pl.ds(pl.multiple_of(v * 16, 16), 16)
            b = ic1[off]
            ord_v[off] = jnp.where(b >= 0, b, ic0[off])
            return c

        lax.fori_loop(0, _NV, mbody, 0)
        pltpu.sync_copy(ord_v, o_ord.at[sl])

        descs = [
            pltpu.async_copy(atom_hbm.at[ord_v], g_atom, sem),
            pltpu.async_copy(pv_hbm.at[ord_v], g_pv, sem),
            pltpu.async_copy(sc_hbm.at[ord_v], g_sc, sem),
            pltpu.async_copy(ex_hbm.at[ord_v], g_ex, sem),
            pltpu.async_copy(re_hbm.at[ord_v], g_re, sem),
        ]
        for dd in descs:
            dd.wait()

        pltpu.sync_copy(g_atom, o_atom.at[sl])
        pltpu.sync_copy(g_pv, o_pv.at[sl])
        pltpu.sync_copy(g_sc, o_sc.at[sl])
        pltpu.sync_copy(g_ex, o_ex.at[sl])
        pltpu.sync_copy(g_re, o_re.at[sl])

    return k(i2, atom, pv, score, ex, react)


def kernel(alignment_tensor, l2_alignment, effective_reactivity, atom_indices,
           refined_peak_points, refined_peak_values, approach_vectors,
           exposure_scores):
    score, mkeys, hist0 = _prep(alignment_tensor, l2_alignment,
                                effective_reactivity)
    hist0 = hist0.reshape(_NW, _RADIX)

    p0 = _pass_kernel(0, first=True, last=False)
    k2, i2, hp = p0(mkeys, hist0)
    p1 = _pass_kernel(8, first=False, last=False)
    k2, i2, hp = p1(k2, i2, hp)
    p2 = _pass_kernel(16, first=False, last=False)
    k2, i2, hp = p2(k2, i2, hp)
    p3 = _pass_kernel(24, first=False, last=True)
    i2 = p3(k2, i2, hp)

    (ranked_atom_indices, psi_peak, alignment_score_ranked, exposure_score,
     effective_reactivity_ranked, order) = _gather_kernel(
        i2, atom_indices, refined_peak_values, score, exposure_scores,
        effective_reactivity)

    som_coordinates = jnp.take(refined_peak_points, order, axis=0,
                               mode="clip")
    approach_vector = jnp.take(approach_vectors, order, axis=0, mode="clip")
    return (ranked_atom_indices, som_coordinates, psi_peak, approach_vector,
            alignment_score_ranked, exposure_score,
            effective_reactivity_ranked)


# speedup vs baseline: 1.6600x; 1.0495x over previous
import functools

import jax
import jax.numpy as jnp
from jax import lax
from jax.experimental import pallas as pl
from jax.experimental.pallas import tpu as pltpu
from jax.experimental.pallas import tpu_sc as plsc

N = 65536
D = 256

_INFO = plsc.get_sparse_core_info()
_NC = _INFO.num_cores
_NS = _INFO.num_subcores
_NW = _NC * _NS
_CPW = N // _NW
_NV = _CPW // 16
_SHARE = N // _NS
_RADIX = 256
_CHUNK = 128

_SC_PARAMS = pltpu.CompilerParams(use_tc_tiling_on_sc=False,
                                  needs_layout_passes=False)


def _mesh():
    return plsc.VectorSubcoreMesh(core_axis_name="c", subcore_axis_name="s")


def _iota16():
    return lax.iota(jnp.int32, 16)


def _srl(v, s):
    return lax.shift_right_logical(v, jnp.full((16,), s, jnp.int32))


def _prep_body(a_ref, l2_ref, score_ref):
    score_ref[...] = 0.5 * jnp.mean(a_ref[...], axis=1) + 0.5 * l2_ref[...]


def _prep(alignment_tensor, l2_alignment):
    R = 2048
    vec = lambda: pl.BlockSpec((R,), lambda i: (i,))
    return pl.pallas_call(
        _prep_body,
        grid=(N // R,),
        in_specs=[pl.BlockSpec((R, D), lambda i: (i, 0)), vec()],
        out_specs=vec(),
        out_shape=jax.ShapeDtypeStruct((N,), jnp.float32),
    )(alignment_tensor, l2_alignment)


def _mono16(r16):
    u = plsc.bitcast(r16, jnp.int32) ^ jnp.int32(-2147483648)
    return jnp.where(u < 0, ~u, u ^ jnp.int32(-2147483648))


def _hist0_kernel(react):
    @functools.partial(
        pl.kernel,
        mesh=_mesh(),
        out_type=jax.ShapeDtypeStruct((_NW, _RADIX), jnp.int32),
        compiler_params=_SC_PARAMS,
        scratch_types=[
            pltpu.VMEM((_CPW,), jnp.float32),
            pltpu.VMEM((16 * _RADIX,), jnp.int32),
            pltpu.VMEM((_RADIX,), jnp.int32),
        ],
    )
    def k(r_hbm, hist_hbm, chunk_v, c2_v, row_v):
        cc = lax.axis_index("c")
        ss = lax.axis_index("s")
        wid = ss * _NC + cc
        pltpu.sync_copy(r_hbm.at[pl.ds(wid * _CPW, _CPW)], chunk_v)
        zero = jnp.zeros((16,), jnp.int32)
        iota = _iota16()
        ones = jnp.ones((16,), jnp.int32)
        lanebase = iota * _RADIX

        def zbody(i, c):
            c2_v[pl.ds(pl.multiple_of(i * 16, 16), 16)] = zero
            return c

        lax.fori_loop(0, 16 * _RADIX // 16, zbody, 0)

        def body(v, c):
            d = _mono16(chunk_v[pl.ds(pl.multiple_of(v * 16, 16), 16)]) & 255
            plsc.addupdate_scatter(c2_v, [lanebase + d], ones)
            return c

        lax.fori_loop(0, _NV, body, 0)

        def rbody(g, c):
            def sbody(l, acc):
                return acc + c2_v[pl.ds(pl.multiple_of(
                    l * _RADIX + g * 16, 16), 16)]

            acc = lax.fori_loop(0, 16, sbody, zero)
            row_v[pl.ds(pl.multiple_of(g * 16, 16), 16)] = acc
            return c

        lax.fori_loop(0, 16, rbody, 0)
        pltpu.sync_copy(row_v, hist_hbm.at[wid])

    return k(react)


def _pass_kernel(shift, first, last):
    nxt = shift + 8

    scratch = [
        pltpu.VMEM((_NW, _RADIX), jnp.int32),
        pltpu.VMEM((_NW, _RADIX), jnp.int32),
        pltpu.VMEM((_CPW,), jnp.float32 if first else jnp.int32),
        pltpu.VMEM((_CPW,), jnp.int32),
        pltpu.VMEM((_CPW,), jnp.int32),
        pltpu.VMEM((_CPW,), jnp.int32),
        pltpu.VMEM((_RADIX,), jnp.int32),
        pltpu.VMEM((_CPW,), jnp.int32),
        pltpu.VMEM((_CPW,), jnp.int32),
        pltpu.VMEM((_CPW // _CHUNK, _CHUNK), jnp.int32),
        pltpu.VMEM((_SHARE,), jnp.int32),
        pltpu.VMEM((_SHARE,), jnp.int32),
        pltpu.VMEM((2 * 16 * _RADIX,), jnp.int32),
        pltpu.VMEM((2, _RADIX), jnp.int32),
        pltpu.VMEM_SHARED((N,), jnp.int32),
        pltpu.VMEM_SHARED((N,), jnp.int32),
        pltpu.SemaphoreType.DMA,
    ]
    if last:
        out_type = jax.ShapeDtypeStruct((_NC, N), jnp.int32)
    else:
        out_type = (jax.ShapeDtypeStruct((_NC, N), jnp.int32),
                    jax.ShapeDtypeStruct((_NC, N), jnp.int32),
                    jax.ShapeDtypeStruct((_NC, _NW, _RADIX), jnp.int32))

    def k(*refs):
        if first:
            k_hbm, h_hbm = refs[:2]
            refs = refs[2:]
        else:
            k_hbm, i_hbm, h_hbm = refs[:3]
            refs = refs[3:]
        if last:
            io_hbm = refs[0]
            refs = refs[1:]
        else:
            ko_hbm, io_hbm, hp_hbm = refs[:3]
            refs = refs[3:]
        (hist_v, hist2_v, kc0, kc1, ic0, ic1, counter_v, kmerged, imerged,
         pos2, islice_v, kslice_v, c2_v, row_v, kshared, ishared, sem) = refs

        cc = lax.axis_index("c")
        ss = lax.axis_index("s")
        wid = ss * _NC + cc
        sl = pl.ds(wid * _CPW, _CPW)
        zero = jnp.zeros((16,), jnp.int32)
        iota = _iota16()

        if first:
            pltpu.sync_copy(h_hbm, hist_v)
            pltpu.sync_copy(k_hbm.at[sl], kc0)
        else:
            pltpu.sync_copy(h_hbm.at[0], hist_v)
            pltpu.sync_copy(h_hbm.at[1], hist2_v)
            pltpu.sync_copy(k_hbm.at[0, sl], kc0)
            pltpu.sync_copy(k_hbm.at[1, sl], kc1)
            pltpu.sync_copy(i_hbm.at[0, sl], ic0)
            pltpu.sync_copy(i_hbm.at[1, sl], ic1)

        negones = jnp.full((16,), -1, jnp.int32)

        def nbody(i, c):
            islice_v[pl.ds(pl.multiple_of(i * 16, 16), 16)] = negones
            return c

        lax.fori_loop(0, _SHARE // 16, nbody, 0)
        my_slice = pl.ds(ss * _SHARE, _SHARE)
        pltpu.sync_copy(islice_v, ishared.at[my_slice])

        def gbody(g, carry):
            go = pl.ds(pl.multiple_of(g * 16, 16), 16)

            def tbody(t, c):
                accg, myg = c
                h = hist_v[t, go]
                if not first:
                    h = h + hist2_v[t, go]
                myg = jnp.where(t == wid, accg, myg)
                return accg + h, myg

            accg, myg = lax.fori_loop(0, _NW, tbody, (zero, zero))
            cs = plsc.cumsum(accg)
            counter_v[go] = (cs - accg) + myg + jnp.broadcast_to(carry, (16,))
            return carry + jnp.sum(accg)

        lax.fori_loop(0, _RADIX // 16, gbody, jnp.int32(0))

        plsc.subcore_barrier()

        def body(v, c):
            off = pl.ds(pl.multiple_of(v * 16, 16), 16)
            if first:
                k16 = _mono16(kc0[off])
                i16 = wid * _CPW + v * 16 + iota
            else:
                i1 = ic1[off]
                msel = i1 >= 0
                k16 = jnp.where(msel, kc1[off], kc0[off])
                i16 = jnp.where(msel, i1, ic0[off])
            d = _srl(k16, shift) & 255
            cnt, lastm = plsc.scan_count(d)
            bases = plsc.load_gather(counter_v, [d])
            pos = bases + cnt - 1
            plsc.store_scatter(counter_v, [d], pos + 1, mask=lastm)
            kmerged[off] = k16
            imerged[off] = i16
            row = lax.shift_right_logical(v, 3)
            col = (v & 7) * 16
            pos2[row, pl.ds(pl.multiple_of(col, 16), 16)] = pos
            return c

        lax.fori_loop(0, _NV, body, 0)

        descs = []
        for j in range(_CPW // _CHUNK):
            sj = pl.ds(j * _CHUNK, _CHUNK)
            if not last:
                descs.append(pltpu.async_copy(kmerged.at[sj],
                                              kshared.at[pos2.at[j]], sem))
            descs.append(pltpu.async_copy(imerged.at[sj],
                                          ishared.at[pos2.at[j]], sem))
        for dd in descs:
            dd.wait()
        plsc.subcore_barrier()

        pltpu.sync_copy(ishared.at[my_slice], islice_v)
        pltpu.sync_copy(islice_v, io_hbm.at[cc, my_slice])
        if last:
            return
        pltpu.sync_copy(kshared.at[my_slice], kslice_v)
        pltpu.sync_copy(kslice_v, ko_hbm.at[cc, my_slice])

        def zbody(i, c):
            c2_v[pl.ds(pl.multiple_of(i * 16, 16), 16)] = zero
            return c

        lax.fori_loop(0, 2 * 16 * _RADIX // 16, zbody, 0)
        lanebase = iota * _RADIX
        ones = jnp.ones((16,), jnp.int32)

        def hbody(v, c):
            off = pl.ds(pl.multiple_of(v * 16, 16), 16)
            k16 = kslice_v[off]
            i16 = islice_v[off]
            valid = i16 >= 0
            d = _srl(k16, nxt) & 255
            half = lax.shift_right_logical(v, 7)
            plsc.addupdate_scatter(c2_v, [half * (16 * _RADIX) + lanebase + d],
                                   ones, mask=valid)
            return c

        lax.fori_loop(0, _SHARE // 16, hbody, 0)

        def rbody(i, c):
            h = lax.shift_right_logical(i, 4)
            g = i & 15

            def sbody(l, acc):
                return acc + c2_v[pl.ds(pl.multiple_of(
                    h * (16 * _RADIX) + l * _RADIX + g * 16, 16), 16)]

            acc = lax.fori_loop(0, 16, sbody, zero)
            row_v[h, pl.ds(pl.multiple_of(g * 16, 16), 16)] = acc
            return c

        lax.fori_loop(0, 32, rbody, 0)
        pltpu.sync_copy(row_v, hp_hbm.at[cc, pl.ds(ss * 2, 2)])

    kern = functools.partial(
        pl.kernel, mesh=_mesh(), out_type=out_type,
        compiler_params=_SC_PARAMS, scratch_types=scratch)(k)
    return kern


def _gather_kernel(i2, atom, pv, score, ex, react):
    out_type = (jax.ShapeDtypeStruct((N,), jnp.int32),
                jax.ShapeDtypeStruct((N,), jnp.float32),
                jax.ShapeDtypeStruct((N,), jnp.float32),
                jax.ShapeDtypeStruct((N,), jnp.float32),
                jax.ShapeDtypeStruct((N,), jnp.float32),
                jax.ShapeDtypeStruct((N,), jnp.int32))

    @functools.partial(
        pl.kernel,
        mesh=_mesh(),
        out_type=out_type,
        compiler_params=_SC_PARAMS,
        scratch_types=[
            pltpu.VMEM((_CPW,), jnp.int32),
            pltpu.VMEM((_CPW,), jnp.int32),
            pltpu.VMEM((_CPW,), jnp.int32),
            pltpu.VMEM((_CPW,), jnp.int32),
            pltpu.VMEM((_CPW,), jnp.float32),
            pltpu.VMEM((_CPW,), jnp.float32),
            pltpu.VMEM((_CPW,), jnp.float32),
            pltpu.VMEM((_CPW,), jnp.float32),
            pltpu.SemaphoreType.DMA,
        ],
    )
    def k(i_hbm, atom_hbm, pv_hbm, sc_hbm, ex_hbm, re_hbm,
          o_atom, o_pv, o_sc, o_ex, o_re, o_ord,
          ic0, ic1, ord_v, g_atom, g_pv, g_sc, g_ex, g_re, sem):
        cc = lax.axis_index("c")
        ss = lax.axis_index("s")
        wid = ss * _NC + cc
        sl = pl.ds(wid * _CPW, _CPW)
        pltpu.sync_copy(i_hbm.at[0, sl], ic0)
        pltpu.sync_copy(i_hbm.at[1, sl], ic1)

        def mbody(v, c):
            off = pl.ds(pl.multiple_of(v * 16, 16), 16)
            b = ic1[off]
            ord_v[off] = jnp.where(b >= 0, b, ic0[off])
            return c

        lax.fori_loop(0, _NV, mbody, 0)
        pltpu.sync_copy(ord_v, o_ord.at[sl])

        descs = [
            pltpu.async_copy(atom_hbm.at[ord_v], g_atom, sem),
            pltpu.async_copy(pv_hbm.at[ord_v], g_pv, sem),
            pltpu.async_copy(sc_hbm.at[ord_v], g_sc, sem),
            pltpu.async_copy(ex_hbm.at[ord_v], g_ex, sem),
            pltpu.async_copy(re_hbm.at[ord_v], g_re, sem),
        ]
        for dd in descs:
            dd.wait()

        pltpu.sync_copy(g_atom, o_atom.at[sl])
        pltpu.sync_copy(g_pv, o_pv.at[sl])
        pltpu.sync_copy(g_sc, o_sc.at[sl])
        pltpu.sync_copy(g_ex, o_ex.at[sl])
        pltpu.sync_copy(g_re, o_re.at[sl])

    return k(i2, atom, pv, score, ex, react)


def kernel(alignment_tensor, l2_alignment, effective_reactivity, atom_indices,
           refined_peak_points, refined_peak_values, approach_vectors,
           exposure_scores):
    score = _prep(alignment_tensor, l2_alignment)
    hist0 = _hist0_kernel(effective_reactivity)

    p0 = _pass_kernel(0, first=True, last=False)
    k2, i2, hp = p0(effective_reactivity, hist0)
    p1 = _pass_kernel(8, first=False, last=False)
    k2, i2, hp = p1(k2, i2, hp)
    p2 = _pass_kernel(16, first=False, last=False)
    k2, i2, hp = p2(k2, i2, hp)
    p3 = _pass_kernel(24, first=False, last=True)
    i2 = p3(k2, i2, hp)

    (ranked_atom_indices, psi_peak, alignment_score_ranked, exposure_score,
     effective_reactivity_ranked, order) = _gather_kernel(
        i2, atom_indices, refined_peak_values, score, exposure_scores,
        effective_reactivity)

    som_coordinates = jnp.take(refined_peak_points, order, axis=0,
                               mode="clip")
    approach_vector = jnp.take(approach_vectors, order, axis=0, mode="clip")
    return (ranked_atom_indices, som_coordinates, psi_peak, approach_vector,
            alignment_score_ranked, exposure_score,
            effective_reactivity_ranked)
